# baseline (speedup 1.0000x reference)
import dataclasses
import functools

import jax
import jax.numpy as jnp
from jax import lax
from jax.experimental import pallas as pl
from jax.experimental.pallas import tpu as pltpu
from jax.experimental.pallas import tpu_sc as plsc

NC = 2
NS = 16
NW = NC * NS
L = 16
CHUNK = 128
NBUF = 4


def _mesh():
    return plsc.VectorSubcoreMesh(core_axis_name="c", subcore_axis_name="s")


def _sc_params():
    cp = pltpu.CompilerParams(use_tc_tiling_on_sc=False)
    if "needs_layout_passes" in pltpu.CompilerParams.__dataclass_fields__:
        cp = dataclasses.replace(cp, needs_layout_passes=False)
    return cp


def _matmul_call(N, D):
    def body(x_ref, w_ref, o_ref):
        o_ref[...] = jnp.dot(x_ref[...], w_ref[...],
                             preferred_element_type=jnp.float32)

    return pl.pallas_call(
        body, out_shape=jax.ShapeDtypeStruct((N, D), jnp.float32)
    )


def _hist_call(Np, EPT):
    cols = Np // NS

    @functools.partial(
        pl.kernel,
        out_type=jax.ShapeDtypeStruct((NC, 2, Np), jnp.float32),
        mesh=_mesh(),
        compiler_params=_sc_params(),
        scratch_types=[
            pltpu.VMEM((Np,), jnp.float32),
            pltpu.VMEM((Np,), jnp.float32),
            pltpu.VMEM((EPT,), jnp.int32),
            pltpu.VMEM((EPT,), jnp.int32),
            pltpu.VMEM((NS, cols), jnp.float32),
            pltpu.VMEM((cols,), jnp.float32),
            pltpu.VMEM_SHARED((NS, Np), jnp.float32),
        ],
    )
    def hist(src_hbm, dst_hbm, zeros_hbm, out_hbm,
             hsrc, hdst, esrc, edst, red, rsum, shared):
        c = lax.axis_index("c")
        s = lax.axis_index("s")
        wid = c * NS + s
        pltpu.sync_copy(zeros_hbm, hsrc)
        pltpu.sync_copy(zeros_hbm, hdst)
        pltpu.sync_copy(src_hbm.at[wid], esrc)
        pltpu.sync_copy(dst_hbm.at[wid], edst)
        ones = jnp.full((L,), 1.0, jnp.float32)

        @pl.loop(0, EPT, step=L)
        def _(j):
            plsc.addupdate_scatter(hsrc, [esrc[pl.ds(j, L)]], ones)
            plsc.addupdate_scatter(hdst, [edst[pl.ds(j, L)]], ones)

        for kind, h in enumerate((hsrc, hdst)):
            pltpu.sync_copy(h, shared.at[s])
            plsc.subcore_barrier()
            pltpu.sync_copy(shared.at[:, pl.ds(s * cols, cols)], red)

            @pl.loop(0, cols, step=L)
            def _(j):
                acc = red[0, pl.ds(j, L)]
                for r in range(1, NS):
                    acc = acc + red[r, pl.ds(j, L)]
                rsum[pl.ds(j, L)] = acc

            pltpu.sync_copy(rsum, out_hbm.at[c, kind, pl.ds(s * cols, cols)])
            plsc.subcore_barrier()

    return hist


def _scale_call(N, Np, D):
    def body(y_ref, h_ref, o_ref):
        deg = h_ref[0, 0, :] + h_ref[1, 0, :]
        s = lax.rsqrt(jnp.maximum(deg[:N], 1.0))
        yb = (y_ref[...] * s[:, None]).astype(jnp.bfloat16)
        for c in range(NC):
            o_ref[c] = yb

    return pl.pallas_call(
        body, out_shape=jax.ShapeDtypeStruct((NC, N, D), jnp.bfloat16)
    )


def _agg_call(Np, CPT, CPT0, D):
    rows_per_tile = Np // NS
    CPT1 = CPT - CPT0

    @functools.partial(
        pl.kernel,
        out_type=jax.ShapeDtypeStruct((NC, Np, D), jnp.bfloat16),
        mesh=_mesh(),
        compiler_params=_sc_params(),
        scratch_types=[
            pltpu.VMEM((CPT, CHUNK), jnp.int32),
            pltpu.VMEM((CPT, CHUNK), jnp.int32),
            [pltpu.VMEM((CHUNK, D), jnp.bfloat16) for _ in range(NBUF)],
            pltpu.VMEM_SHARED((Np, D), jnp.bfloat16),
            [pltpu.SemaphoreType.DMA for _ in range(NBUF)],
            [pltpu.SemaphoreType.DMA for _ in range(NBUF)],
        ],
    )
    def agg(feat_hbm, src_hbm, dst_hbm, zrows_hbm, out_hbm,
            sidx, didx, bufs, accum, gsems, ssems):
        c = lax.axis_index("c")
        s = lax.axis_index("s")
        base = jnp.where(c == 0, 0, CPT0)
        nch = jnp.where(c == 0, CPT0, CPT1)
        my_rows = pl.ds(s * rows_per_tile, rows_per_tile)
        pltpu.sync_copy(zrows_hbm.at[my_rows], accum.at[my_rows])
        pltpu.sync_copy(src_hbm.at[s], sidx)
        pltpu.sync_copy(dst_hbm.at[s], didx)
        plsc.subcore_barrier()
        feat_c = feat_hbm.at[c]

        def wait_gather(b, ch):
            pltpu.make_async_copy(feat_c.at[sidx.at[ch]], bufs[b],
                                  gsems[b]).wait()

        def scatter_add(b, ch):
            pltpu.async_copy(bufs[b], accum.at[didx.at[ch]], ssems[b],
                             add=True).wait()

        for b in range(NBUF):
            pltpu.async_copy(feat_c.at[sidx.at[base + b]], bufs[b],
                             gsems[b])

        @pl.loop(0, nch - NBUF, step=NBUF)
        def _(c0):
            for b in range(NBUF):
                ch = base + c0 + b
                wait_gather(b, ch)
                scatter_add(b, ch)
                pltpu.async_copy(feat_c.at[sidx.at[ch + NBUF]], bufs[b],
                                 gsems[b])

        for b in range(NBUF):
            ch = base + nch - NBUF + b
            wait_gather(b, ch)
            scatter_add(b, ch)

        plsc.subcore_barrier()
        pltpu.sync_copy(accum.at[my_rows], out_hbm.at[c].at[my_rows])

    return agg


def _out_call(Np, D):
    def body(a_ref, h_ref, b_ref, o_ref):
        a = a_ref[0].astype(jnp.float32) + a_ref[1].astype(jnp.float32)
        deg = h_ref[0, 1, :] + h_ref[1, 1, :]
        s = lax.rsqrt(jnp.maximum(deg, 1.0))
        o_ref[...] = jnp.maximum(a * s[:, None] + b_ref[...], 0.0)

    return pl.pallas_call(
        body, out_shape=jax.ShapeDtypeStruct((Np, D), jnp.float32)
    )


def kernel(features, edge_index, W, b):
    N, D = features.shape
    E = edge_index.shape[1]
    Np = -(-(N + 1) // 2048) * 2048
    grain = CHUNK * NBUF
    EPT = -(-E // (NS * grain)) * grain
    CPT = EPT // CHUNK
    CPT0 = max(NBUF, min(CPT - NBUF, (-(-(CPT * 3) // 4) // NBUF) * NBUF))
    E_pad = NS * EPT

    srcp = jnp.concatenate([edge_index[0],
                            jnp.zeros((E_pad - E,), jnp.int32)])
    dstp = jnp.concatenate([edge_index[1],
                            jnp.full((E_pad - E,), N, jnp.int32)])
    zeros1 = jnp.zeros((Np,), jnp.float32)
    zeros2 = jnp.zeros((Np, D), jnp.bfloat16)

    y = _matmul_call(N, D)(features.astype(jnp.float32),
                           W.astype(jnp.float32))
    if E % (NW * L) == 0:
        hsrc, hdst, EPW = edge_index[0], edge_index[1], E // NW
    else:
        padN = jnp.full((-(-E // (NW * L)) * NW * L - E,), N, jnp.int32)
        hsrc = jnp.concatenate([edge_index[0], padN])
        hdst = jnp.concatenate([edge_index[1], padN])
        EPW = hsrc.shape[0] // NW
    hists = _hist_call(Np, EPW)(hsrc.reshape(NW, EPW), hdst.reshape(NW, EPW),
                                zeros1)
    yb = _scale_call(N, Np, D)(y, hists)
    agg = _agg_call(Np, CPT, CPT0, D)(yb, srcp.reshape(NS, CPT, CHUNK),
                                      dstp.reshape(NS, CPT, CHUNK), zeros2)
    out = _out_call(Np, D)(agg, hists, b.astype(jnp.float32).reshape(1, D))
    return out[:N]


# speedup vs baseline: 1.0519x; 1.0519x over previous
import dataclasses
import functools

import jax
import jax.numpy as jnp
from jax import lax
from jax.experimental import pallas as pl
from jax.experimental.pallas import tpu as pltpu
from jax.experimental.pallas import tpu_sc as plsc

NC = 2
NS = 16
NW = NC * NS
L = 16
CHUNK = 128
NBUF = 4


def _mesh():
    return plsc.VectorSubcoreMesh(core_axis_name="c", subcore_axis_name="s")


def _sc_params():
    cp = pltpu.CompilerParams(use_tc_tiling_on_sc=False)
    if "needs_layout_passes" in pltpu.CompilerParams.__dataclass_fields__:
        cp = dataclasses.replace(cp, needs_layout_passes=False)
    return cp


def _matmul_call(N, D):
    def body(x_ref, w_ref, o_ref):
        o_ref[...] = jnp.dot(x_ref[...], w_ref[...],
                             preferred_element_type=jnp.float32)

    return pl.pallas_call(
        body, out_shape=jax.ShapeDtypeStruct((N, D), jnp.float32)
    )


def _hist_call(Np, EPT):
    cols = Np // NS

    @functools.partial(
        pl.kernel,
        out_type=jax.ShapeDtypeStruct((NC, 2, Np), jnp.float32),
        mesh=_mesh(),
        compiler_params=_sc_params(),
        scratch_types=[
            pltpu.VMEM((Np,), jnp.float32),
            pltpu.VMEM((Np,), jnp.float32),
            pltpu.VMEM((EPT,), jnp.int32),
            pltpu.VMEM((EPT,), jnp.int32),
            pltpu.VMEM((NS, cols), jnp.float32),
            pltpu.VMEM((cols,), jnp.float32),
            pltpu.VMEM_SHARED((NS, Np), jnp.float32),
        ],
    )
    def hist(src_hbm, dst_hbm, zeros_hbm, out_hbm,
             hsrc, hdst, esrc, edst, red, rsum, shared):
        c = lax.axis_index("c")
        s = lax.axis_index("s")
        wid = c * NS + s
        pltpu.sync_copy(zeros_hbm, hsrc)
        pltpu.sync_copy(zeros_hbm, hdst)
        pltpu.sync_copy(src_hbm.at[wid], esrc)
        pltpu.sync_copy(dst_hbm.at[wid], edst)
        ones = jnp.full((L,), 1.0, jnp.float32)

        @pl.loop(0, EPT, step=L)
        def _(j):
            plsc.addupdate_scatter(hsrc, [esrc[pl.ds(j, L)]], ones)
            plsc.addupdate_scatter(hdst, [edst[pl.ds(j, L)]], ones)

        for kind, h in enumerate((hsrc, hdst)):
            pltpu.sync_copy(h, shared.at[s])
            plsc.subcore_barrier()
            pltpu.sync_copy(shared.at[:, pl.ds(s * cols, cols)], red)

            @pl.loop(0, cols, step=L)
            def _(j):
                acc = red[0, pl.ds(j, L)]
                for r in range(1, NS):
                    acc = acc + red[r, pl.ds(j, L)]
                rsum[pl.ds(j, L)] = acc

            pltpu.sync_copy(rsum, out_hbm.at[c, kind, pl.ds(s * cols, cols)])
            plsc.subcore_barrier()

    return hist


def _scale_call(N, Np, D):
    def body(y_ref, h_ref, o_ref):
        deg = h_ref[0, 0, :] + h_ref[1, 0, :]
        s = lax.rsqrt(jnp.maximum(deg[:N], 1.0))
        yb = (y_ref[...] * s[:, None]).astype(jnp.bfloat16)
        for c in range(NC):
            o_ref[c] = yb

    return pl.pallas_call(
        body, out_shape=jax.ShapeDtypeStruct((NC, N, D), jnp.bfloat16)
    )


def _agg_call(Np, CPT, CPT0, D):
    rows_per_tile = Np // NS
    CPT1 = CPT - CPT0

    @functools.partial(
        pl.kernel,
        out_type=jax.ShapeDtypeStruct((NC, Np, D), jnp.bfloat16),
        mesh=_mesh(),
        compiler_params=_sc_params(),
        scratch_types=[
            pltpu.VMEM((CPT, CHUNK), jnp.int32),
            pltpu.VMEM((CPT, CHUNK), jnp.int32),
            [pltpu.VMEM((CHUNK, D), jnp.bfloat16) for _ in range(NBUF)],
            pltpu.VMEM_SHARED((Np, D), jnp.bfloat16),
            [pltpu.SemaphoreType.DMA for _ in range(NBUF)],
            [pltpu.SemaphoreType.DMA for _ in range(NBUF)],
        ],
    )
    def agg(feat_hbm, src_hbm, dst_hbm, zrows_hbm, out_hbm,
            sidx, didx, bufs, accum, gsems, ssems):
        c = lax.axis_index("c")
        s = lax.axis_index("s")
        base = jnp.where(c == 0, 0, CPT0)
        nch = jnp.where(c == 0, CPT0, CPT1)
        my_rows = pl.ds(s * rows_per_tile, rows_per_tile)
        pltpu.sync_copy(zrows_hbm.at[my_rows], accum.at[my_rows])
        pltpu.sync_copy(src_hbm.at[s], sidx)
        pltpu.sync_copy(dst_hbm.at[s], didx)
        plsc.subcore_barrier()
        feat_c = feat_hbm.at[c]

        def wait_gather(b, ch):
            pltpu.make_async_copy(feat_c.at[sidx.at[ch]], bufs[b],
                                  gsems[b]).wait()

        def scatter_add(b, ch):
            pltpu.async_copy(bufs[b], accum.at[didx.at[ch]], ssems[b],
                             add=True).wait()

        for b in range(NBUF):
            pltpu.async_copy(feat_c.at[sidx.at[base + b]], bufs[b],
                             gsems[b])

        @pl.loop(0, nch - NBUF, step=NBUF)
        def _(c0):
            for b in range(NBUF):
                ch = base + c0 + b
                wait_gather(b, ch)
                scatter_add(b, ch)
                pltpu.async_copy(feat_c.at[sidx.at[ch + NBUF]], bufs[b],
                                 gsems[b])

        for b in range(NBUF):
            ch = base + nch - NBUF + b
            wait_gather(b, ch)
            scatter_add(b, ch)

        plsc.subcore_barrier()
        pltpu.sync_copy(accum.at[my_rows], out_hbm.at[c].at[my_rows])

    return agg


def _out_call(Np, D):
    def body(a_ref, h_ref, b_ref, o_ref):
        a = a_ref[0].astype(jnp.float32) + a_ref[1].astype(jnp.float32)
        deg = h_ref[0, 1, :] + h_ref[1, 1, :]
        s = lax.rsqrt(jnp.maximum(deg, 1.0))
        o_ref[...] = jnp.maximum(a * s[:, None] + b_ref[...], 0.0)

    return pl.pallas_call(
        body, out_shape=jax.ShapeDtypeStruct((Np, D), jnp.float32)
    )


def kernel(features, edge_index, W, b):
    N, D = features.shape
    E = edge_index.shape[1]
    Np = -(-(N + 1) // 2048) * 2048
    grain = CHUNK * NBUF
    EPT = -(-E // (NS * grain)) * grain
    CPT = EPT // CHUNK
    CPT0 = max(NBUF, min(CPT - NBUF, (-(-(CPT * 31) // 40) // NBUF) * NBUF))
    E_pad = NS * EPT

    srcp = jnp.concatenate([edge_index[0],
                            jnp.zeros((E_pad - E,), jnp.int32)])
    dstp = jnp.concatenate([edge_index[1],
                            jnp.full((E_pad - E,), N, jnp.int32)])
    zeros1 = jnp.zeros((Np,), jnp.float32)
    zeros2 = jnp.zeros((Np, D), jnp.bfloat16)

    y = _matmul_call(N, D)(features.astype(jnp.float32),
                           W.astype(jnp.float32))
    if E % (NW * L) == 0:
        hsrc, hdst, EPW = edge_index[0], edge_index[1], E // NW
    else:
        padN = jnp.full((-(-E // (NW * L)) * NW * L - E,), N, jnp.int32)
        hsrc = jnp.concatenate([edge_index[0], padN])
        hdst = jnp.concatenate([edge_index[1], padN])
        EPW = hsrc.shape[0] // NW
    hists = _hist_call(Np, EPW)(hsrc.reshape(NW, EPW), hdst.reshape(NW, EPW),
                                zeros1)
    yb = _scale_call(N, Np, D)(y, hists)
    agg = _agg_call(Np, CPT, CPT0, D)(yb, srcp.reshape(NS, CPT, CHUNK),
                                      dstp.reshape(NS, CPT, CHUNK), zeros2)
    out = _out_call(Np, D)(agg, hists, b.astype(jnp.float32).reshape(1, D))
    return out[:N]
